# baseline (speedup 1.0000x reference)
import functools

import jax
import jax.numpy as jnp
from jax import lax
from jax.experimental import pallas as pl
from jax.experimental.pallas import tpu as pltpu
from jax.experimental.pallas import tpu_sc as plsc

N = 10000
D = 256
G = 64
NC = 2
NS = 16
NW = NC * NS
L = 16
ROWS_W = 320
CH = 128
NCHUNK = 3
VROWS = NCHUNK * CH
GROUPS_CH = CH // L
SROWS = 80
TAIL_W = NW - 1
TAIL_ROWS = N - TAIL_W * ROWS_W
FULL_SIZES = (CH, CH, ROWS_W - 2 * CH)
TAIL_SIZES = (TAIL_ROWS, 0, 0)


def _x_copy(x_hbm, xbuf, sem, base, j, nrows):
    return pltpu.make_async_copy(
        x_hbm.at[pl.ds(base + j * CH, nrows)],
        xbuf.at[pl.ds(j * CH, nrows)], sem)


def _sc_body(x_hbm, b_hbm, xsum_hbm, xbuf, bbuf, table, *sems):
    c = lax.axis_index("c")
    s = lax.axis_index("s")
    w = c * NS + s
    base = w * ROWS_W

    @pl.when(w < TAIL_W)
    def _():
        for j, n in enumerate(FULL_SIZES):
            _x_copy(x_hbm, xbuf, sems[j], base, j, n).start()

    @pl.when(w == TAIL_W)
    def _():
        for j, n in enumerate(TAIL_SIZES):
            if n:
                _x_copy(x_hbm, xbuf, sems[j], base, j, n).start()

    gvec = jnp.full((L,), G, jnp.int32)
    for k in range(VROWS // L):
        bbuf[pl.ds(k * L, L)] = gvec

    @pl.when(w < TAIL_W)
    def _():
        pltpu.sync_copy(b_hbm.at[pl.ds(base, ROWS_W)],
                        bbuf.at[pl.ds(0, ROWS_W)])

    @pl.when(w == TAIL_W)
    def _():
        pltpu.sync_copy(b_hbm.at[pl.ds(base, TAIL_ROWS)],
                        bbuf.at[pl.ds(0, TAIL_ROWS)])

    zvec = jnp.zeros((L,), jnp.float32)
    for r in range(SROWS):
        table[r, pl.ds(0, L)] = zvec

    def group_body(q, _):
        bq = bbuf[pl.ds(q * L, L)]
        for i in range(L):
            r = q * L + i
            acc = xbuf[r, pl.ds(0, L)]
            for k in range(1, D // L):
                acc = acc + xbuf[r, pl.ds(k * L, L)]
            b = bq[i]
            table[b, pl.ds(0, L)] = table[b, pl.ds(0, L)] + acc
        return 0

    for j in range(NCHUNK):
        @pl.when(w < TAIL_W)
        def _():
            _x_copy(x_hbm, xbuf, sems[j], base, j, FULL_SIZES[j]).wait()

        if TAIL_SIZES[j]:
            @pl.when(w == TAIL_W)
            def _():
                _x_copy(x_hbm, xbuf, sems[j], base, j, TAIL_SIZES[j]).wait()

        lax.fori_loop(j * GROUPS_CH, (j + 1) * GROUPS_CH, group_body, 0)

    pltpu.sync_copy(table, xsum_hbm.at[c].at[s])


@jax.jit
def _sc_segment(x, b):
    mesh = plsc.VectorSubcoreMesh(core_axis_name="c", subcore_axis_name="s",
                                  num_cores=NC, num_subcores=NS)
    return pl.kernel(
        _sc_body,
        out_type=jax.ShapeDtypeStruct((NC, NS, SROWS, 128), jnp.float32),
        mesh=mesh,
        scratch_types=[
            pltpu.VMEM((VROWS, D), jnp.float32),
            pltpu.VMEM((VROWS,), jnp.int32),
            pltpu.VMEM((SROWS, 128), jnp.float32),
        ] + [pltpu.SemaphoreType.DMA] * NCHUNK,
    )(x, b)


NPAD = NW * ROWS_W


def _tc_combine_body(xsum_ref, u_ref, bcol_ref, out_ref):
    t = jnp.sum(xsum_ref[...], axis=(0, 1))
    tot = jnp.sum(t[:G, :L], axis=1)
    bb = jnp.broadcast_to(bcol_ref[...], (NPAD, G))
    gi = lax.broadcasted_iota(jnp.int32, (NPAD, G), 1)
    counts = jnp.sum((bb == gi).astype(jnp.float32), axis=0)
    usum = jnp.sum(u_ref[...], axis=1)
    out_ref[...] = usum + tot / jnp.maximum(counts, 1.0)


@jax.jit
def _tc_combine(xsum, u, bcol):
    return pl.pallas_call(
        _tc_combine_body,
        out_shape=jax.ShapeDtypeStruct((G,), jnp.float32),
    )(xsum, u, bcol)


def kernel(x, edge_index, edge_attr, u, batch):
    del edge_index, edge_attr
    b = batch.astype(jnp.int32)
    xsum = _sc_segment(x, b)
    bcol = jnp.pad(b, (0, NPAD - N), constant_values=G)[:, None]
    return _tc_combine(xsum, u, bcol)


# speedup vs baseline: 1.0655x; 1.0655x over previous
import functools

import jax
import jax.numpy as jnp
from jax import lax
from jax.experimental import pallas as pl
from jax.experimental.pallas import tpu as pltpu
from jax.experimental.pallas import tpu_sc as plsc

N = 10000
D = 256
G = 64
NC = 2
NS = 16
NW = NC * NS
L = 16
ROWS_W = 320
CH = 128
NCHUNK = 3
VROWS = NCHUNK * CH
GROUPS_CH = CH // L
SROWS = 160
TAIL_W = NW - 1
TAIL_ROWS = N - TAIL_W * ROWS_W
FULL_SIZES = (CH, CH, ROWS_W - 2 * CH)
TAIL_SIZES = (TAIL_ROWS, 0, 0)


def _x_copy(x_hbm, xbuf, sem, base, j, nrows):
    return pltpu.make_async_copy(
        x_hbm.at[pl.ds(base + j * CH, nrows)],
        xbuf.at[pl.ds(j * CH, nrows)], sem)


def _sc_body(x_hbm, b_hbm, xsum_hbm, xbuf, bbuf, table, *sems):
    c = lax.axis_index("c")
    s = lax.axis_index("s")
    w = c * NS + s
    base = w * ROWS_W

    @pl.when(w < TAIL_W)
    def _():
        for j, n in enumerate(FULL_SIZES):
            _x_copy(x_hbm, xbuf, sems[j], base, j, n).start()

    @pl.when(w == TAIL_W)
    def _():
        for j, n in enumerate(TAIL_SIZES):
            if n:
                _x_copy(x_hbm, xbuf, sems[j], base, j, n).start()

    gvec = jnp.full((L,), G, jnp.int32)
    for k in range(VROWS // L):
        bbuf[pl.ds(k * L, L)] = gvec

    @pl.when(w < TAIL_W)
    def _():
        pltpu.sync_copy(b_hbm.at[pl.ds(base, ROWS_W)],
                        bbuf.at[pl.ds(0, ROWS_W)])

    @pl.when(w == TAIL_W)
    def _():
        pltpu.sync_copy(b_hbm.at[pl.ds(base, TAIL_ROWS)],
                        bbuf.at[pl.ds(0, TAIL_ROWS)])

    zvec = jnp.zeros((L,), jnp.float32)
    for r in range(SROWS):
        table[r, pl.ds(0, L)] = zvec

    onev = jnp.ones((L,), jnp.float32)
    fullv = jnp.full((L,), float(L), jnp.float32)

    def group_body(q, _):
        bq = bbuf[pl.ds(q * L, L)]
        b0 = bq[0]
        blast = bq[L - 1]

        @pl.when(b0 == blast)
        def _():
            parts = []
            for i in range(L):
                r = q * L + i
                a = xbuf[r, pl.ds(0, L)]
                for k in range(1, D // L):
                    a = a + xbuf[r, pl.ds(k * L, L)]
                parts.append(a)
            while len(parts) > 1:
                parts = [parts[i] + parts[i + 1]
                         for i in range(0, len(parts), 2)]
            table[b0, pl.ds(0, L)] = table[b0, pl.ds(0, L)] + parts[0]
            bc = b0 + (SROWS // 2)
            table[bc, pl.ds(0, L)] = table[bc, pl.ds(0, L)] + fullv

        @pl.when(b0 != blast)
        def _():
            for i in range(L):
                r = q * L + i
                acc = xbuf[r, pl.ds(0, L)]
                for k in range(1, D // L):
                    acc = acc + xbuf[r, pl.ds(k * L, L)]
                b = bq[i]
                table[b, pl.ds(0, L)] = table[b, pl.ds(0, L)] + acc
                bc = b + (SROWS // 2)
                table[bc, pl.ds(0, L)] = table[bc, pl.ds(0, L)] + onev
        return 0

    for j in range(NCHUNK):
        @pl.when(w < TAIL_W)
        def _():
            _x_copy(x_hbm, xbuf, sems[j], base, j, FULL_SIZES[j]).wait()

        if TAIL_SIZES[j]:
            @pl.when(w == TAIL_W)
            def _():
                _x_copy(x_hbm, xbuf, sems[j], base, j, TAIL_SIZES[j]).wait()

        lax.fori_loop(j * GROUPS_CH, (j + 1) * GROUPS_CH, group_body, 0)

    pltpu.sync_copy(table, xsum_hbm.at[c].at[s])


@jax.jit
def _sc_segment(x, b):
    mesh = plsc.VectorSubcoreMesh(core_axis_name="c", subcore_axis_name="s",
                                  num_cores=NC, num_subcores=NS)
    return pl.kernel(
        _sc_body,
        out_type=jax.ShapeDtypeStruct((NC, NS, SROWS, 128), jnp.float32),
        mesh=mesh,
        scratch_types=[
            pltpu.VMEM((VROWS, D), jnp.float32),
            pltpu.VMEM((VROWS,), jnp.int32),
            pltpu.VMEM((SROWS, 128), jnp.float32),
        ] + [pltpu.SemaphoreType.DMA] * NCHUNK,
    )(x, b)


def _tc_combine_body(xsum_ref, u_ref, out_ref):
    t = jnp.sum(xsum_ref[...], axis=(0, 1))
    tot = jnp.sum(t[:G, :L], axis=1)
    counts = jnp.sum(t[SROWS // 2:SROWS // 2 + G, :L], axis=1) * (1.0 / L)
    usum = jnp.sum(u_ref[...], axis=1)
    out_ref[...] = usum + tot / jnp.maximum(counts, 1.0)


@jax.jit
def _tc_combine(xsum, u):
    return pl.pallas_call(
        _tc_combine_body,
        out_shape=jax.ShapeDtypeStruct((G,), jnp.float32),
    )(xsum, u)


def kernel(x, edge_index, edge_attr, u, batch):
    del edge_index, edge_attr
    b = batch.astype(jnp.int32)
    xsum = _sc_segment(x, b)
    return _tc_combine(xsum, u)
